# Initial kernel scaffold; baseline (speedup 1.0000x reference)
import functools

import jax
import jax.numpy as jnp
from jax import lax
from jax.experimental import pallas as pl
from jax.experimental.pallas import tpu as pltpu
from jax.experimental.pallas import tpu_sc as plsc

N = 10000
E = 320000
D = 128

NC = 2
NS = 16
NW = NC * NS
E_PER_W = E // NW
CHUNK = 80
N_CHUNKS = E_PER_W // CHUNK
ROWS_PER_TILE = N // NS


def _sc_spmm(row, col, val, embeds, zeros_blk):
    mesh = plsc.VectorSubcoreMesh(core_axis_name="c", subcore_axis_name="s")

    @functools.partial(
        pl.kernel,
        out_type=jax.ShapeDtypeStruct((NC, N, D), jnp.float32),
        mesh=mesh,
        scratch_types=[
            pltpu.VMEM_SHARED((N, D), jnp.float32),
            pltpu.VMEM((CHUNK,), jnp.int32),
            pltpu.VMEM((CHUNK,), jnp.int32),
            pltpu.VMEM((CHUNK,), jnp.float32),
            pltpu.VMEM((CHUNK, D), jnp.float32),
            pltpu.SemaphoreType.DMA,
        ],
    )
    def k(row_h, col_h, val_h, emb_h, zero_h, out_h, acc, colv, rowv, valv, rows, sem):
        cid = lax.axis_index("c")
        sid = lax.axis_index("s")
        wid = cid * NS + sid

        pltpu.sync_copy(zero_h, acc.at[pl.ds(sid * ROWS_PER_TILE, ROWS_PER_TILE)])
        plsc.subcore_barrier()

        def chunk_body(ci, carry):
            base = wid * E_PER_W + ci * CHUNK
            pltpu.sync_copy(col_h.at[pl.ds(base, CHUNK)], colv)
            pltpu.sync_copy(row_h.at[pl.ds(base, CHUNK)], rowv)
            pltpu.sync_copy(val_h.at[pl.ds(base, CHUNK)], valv)
            pltpu.async_copy(emb_h.at[colv], rows, sem).wait()

            def e_body(e, ecarry):
                v = valv[e]
                for j in range(D // 16):
                    sl = pl.ds(j * 16, 16)
                    rows[e, sl] = rows[e, sl] * v
                return ecarry

            lax.fori_loop(0, CHUNK, e_body, 0)

            pltpu.sync_copy(rows, acc.at[rowv], add=True)
            return carry

        lax.fori_loop(0, N_CHUNKS, chunk_body, 0)

        plsc.subcore_barrier()
        sl = pl.ds(sid * ROWS_PER_TILE, ROWS_PER_TILE)
        pltpu.sync_copy(acc.at[sl], out_h.at[cid, sl])

    return k(row, col, val, embeds, zeros_blk)


def _tc_add(partials):
    def body(p_ref, o_ref):
        o_ref[...] = p_ref[0] + p_ref[1]

    return pl.pallas_call(
        body,
        out_shape=jax.ShapeDtypeStruct((N, D), jnp.float32),
        grid=(8,),
        in_specs=[pl.BlockSpec((NC, N // 8, D), lambda i: (0, i, 0))],
        out_specs=pl.BlockSpec((N // 8, D), lambda i: (i, 0)),
    )(partials)


def kernel(edge_index, edge_values, embeds):
    row = edge_index[0]
    col = edge_index[1]
    zeros_blk = jnp.zeros((ROWS_PER_TILE, D), jnp.float32)
    partials = _sc_spmm(row, col, edge_values, embeds, zeros_blk)
    return _tc_add(partials)


# speedup vs baseline: 4.4994x; 4.4994x over previous
import functools

import jax
import jax.numpy as jnp
from jax import lax
from jax.experimental import pallas as pl
from jax.experimental.pallas import tpu as pltpu
from jax.experimental.pallas import tpu_sc as plsc

N = 10000
E = 320000
D = 128

NC = 2
NS = 16
NW = NC * NS
E_PER_W = E // NW
CHUNK = 80
N_CHUNKS = E_PER_W // CHUNK
WB_TILES = 10
WB_ROWS = N // WB_TILES


def _sc_spmm(row, col, val, embeds, zeros_blk):
    mesh = plsc.VectorSubcoreMesh(core_axis_name="c", subcore_axis_name="s")

    @functools.partial(
        pl.kernel,
        out_type=jax.ShapeDtypeStruct((NC, N, D), jnp.float32),
        mesh=mesh,
        scratch_types=[
            pltpu.VMEM_SHARED((N, D), jnp.float32),
            pltpu.VMEM((CHUNK,), jnp.int32),
            pltpu.VMEM((CHUNK,), jnp.int32),
            pltpu.VMEM((CHUNK,), jnp.float32),
            pltpu.VMEM((CHUNK, D), jnp.float32),
            pltpu.SemaphoreType.DMA,
        ],
    )
    def k(row_h, col_h, val_h, emb_h, zero_h, out_h, acc, colv, rowv, valv, rows, sem):
        cid = lax.axis_index("c")
        sid = lax.axis_index("s")
        wid = cid * NS + sid

        @pl.when(sid < WB_TILES)
        def _():
            pltpu.sync_copy(zero_h, acc.at[pl.ds(sid * WB_ROWS, WB_ROWS)])

        plsc.subcore_barrier()

        def chunk_body(ci, carry):
            base = wid * E_PER_W + ci * CHUNK
            pltpu.sync_copy(col_h.at[pl.ds(base, CHUNK)], colv)
            pltpu.sync_copy(row_h.at[pl.ds(base, CHUNK)], rowv)
            pltpu.sync_copy(val_h.at[pl.ds(base, CHUNK)], valv)
            pltpu.async_copy(emb_h.at[colv], rows, sem).wait()

            def g_body(g, ecarry):
                vv = valv[pl.ds(g * 16, 16)]
                for t in range(16):
                    v = vv[t]
                    e = g * 16 + t
                    for j in range(D // 16):
                        sl = pl.ds(j * 16, 16)
                        rows[e, sl] = rows[e, sl] * v
                return ecarry

            lax.fori_loop(0, CHUNK // 16, g_body, 0)

            pltpu.sync_copy(rows, acc.at[rowv], add=True)
            return carry

        lax.fori_loop(0, N_CHUNKS, chunk_body, 0)

        plsc.subcore_barrier()

        @pl.when(sid < WB_TILES)
        def _():
            sl = pl.ds(sid * WB_ROWS, WB_ROWS)
            pltpu.sync_copy(acc.at[sl], out_h.at[cid, sl])

    return k(row, col, val, embeds, zeros_blk)


def _tc_add(partials):
    def body(p_ref, o_ref):
        o_ref[...] = p_ref[0] + p_ref[1]

    return pl.pallas_call(
        body,
        out_shape=jax.ShapeDtypeStruct((N, D), jnp.float32),
        grid=(10,),
        in_specs=[pl.BlockSpec((NC, N // 10, D), lambda i: (0, i, 0))],
        out_specs=pl.BlockSpec((N // 10, D), lambda i: (i, 0)),
    )(partials)


def kernel(edge_index, edge_values, embeds):
    row = edge_index[0]
    col = edge_index[1]
    zeros_blk = jnp.zeros((WB_ROWS, D), jnp.float32)
    partials = _sc_spmm(row, col, edge_values, embeds, zeros_blk)
    return _tc_add(partials)


# speedup vs baseline: 9.7667x; 2.1707x over previous
import functools

import jax
import jax.numpy as jnp
from jax import lax
from jax.experimental import pallas as pl
from jax.experimental.pallas import tpu as pltpu
from jax.experimental.pallas import tpu_sc as plsc

N = 10000
E = 320000
D = 128

NC = 2
NS = 16
NW = NC * NS
E_PER_W = E // NW
CHUNK = 80
N_CHUNKS = E_PER_W // CHUNK
N_PAIRS = N_CHUNKS // 2
WB_TILES = 10
WB_ROWS = N // WB_TILES


def _sc_spmm(packed3, val3, embeds, zeros_blk):
    mesh = plsc.VectorSubcoreMesh(core_axis_name="c", subcore_axis_name="s")

    @functools.partial(
        pl.kernel,
        out_type=jax.ShapeDtypeStruct((NC, N, D), jnp.float32),
        mesh=mesh,
        scratch_types=[
            pltpu.VMEM_SHARED((N, D), jnp.float32),
            pltpu.VMEM((E_PER_W,), jnp.int32),
            pltpu.VMEM((E_PER_W,), jnp.float32),
            pltpu.VMEM((CHUNK,), jnp.int32),
            pltpu.VMEM((CHUNK,), jnp.int32),
            pltpu.VMEM((CHUNK,), jnp.int32),
            pltpu.VMEM((CHUNK,), jnp.int32),
            pltpu.VMEM((CHUNK, D), jnp.float32),
            pltpu.VMEM((CHUNK, D), jnp.float32),
            pltpu.SemaphoreType.DMA,
            pltpu.SemaphoreType.DMA,
            pltpu.SemaphoreType.DMA,
            pltpu.SemaphoreType.DMA,
        ],
    )
    def k(packed_h, val_h, emb_h, zero_h, out_h,
          acc, packed, vals, colb0, colb1, rowb0, rowb1, buf0, buf1,
          gs0, gs1, ss0, ss1):
        cid = lax.axis_index("c")
        sid = lax.axis_index("s")
        wid = cid * NS + sid

        pltpu.sync_copy(packed_h.at[wid], packed)
        pltpu.sync_copy(val_h.at[wid], vals)

        @pl.when(sid < WB_TILES)
        def _():
            pltpu.sync_copy(zero_h, acc.at[pl.ds(sid * WB_ROWS, WB_ROWS)])

        plsc.subcore_barrier()

        def unpack(ci, colb, rowb):
            for g in range(CHUNK // 16):
                sl = pl.ds(g * 16, 16)
                p = packed[pl.ds(ci * CHUNK + g * 16, 16)]
                colb[sl] = lax.bitwise_and(p, 0xFFFF)
                rowb[sl] = lax.shift_right_logical(p, 16)

        def gather_start(buf, colb, sem):
            pltpu.async_copy(emb_h.at[colb], buf, sem)

        def gather_wait(buf, colb, sem):
            pltpu.make_async_copy(emb_h.at[colb], buf, sem).wait()

        def scatter_start(buf, rowb, sem):
            pltpu.async_copy(buf, acc.at[rowb], sem, add=True)

        def scatter_wait(buf, rowb, sem):
            pltpu.make_async_copy(buf, acc.at[rowb], sem).wait()

        def scale(buf, ci):
            for g in range(CHUNK // 16):
                vv = vals[pl.ds(ci * CHUNK + g * 16, 16)]
                for t in range(16):
                    v = vv[t]
                    e = g * 16 + t
                    for j in range(D // 16):
                        sl = pl.ds(j * 16, 16)
                        buf[e, sl] = buf[e, sl] * v

        unpack(0, colb0, rowb0)
        gather_start(buf0, colb0, gs0)
        unpack(1, colb1, rowb1)
        gather_start(buf1, colb1, gs1)

        def pair_body(i, carry):
            c0 = 2 * i
            c1 = 2 * i + 1
            gather_wait(buf0, colb0, gs0)
            scale(buf0, c0)
            scatter_start(buf0, rowb0, ss0)

            gather_wait(buf1, colb1, gs1)
            scale(buf1, c1)
            scatter_start(buf1, rowb1, ss1)

            scatter_wait(buf0, rowb0, ss0)
            unpack(c0 + 2, colb0, rowb0)
            gather_start(buf0, colb0, gs0)

            @pl.when(i < N_PAIRS - 1)
            def _():
                scatter_wait(buf1, rowb1, ss1)
                unpack(c1 + 2, colb1, rowb1)
                gather_start(buf1, colb1, gs1)

            return carry

        lax.fori_loop(0, N_PAIRS, pair_body, 0)

        last = N_CHUNKS - 1
        gather_wait(buf0, colb0, gs0)
        scale(buf0, last)
        scatter_start(buf0, rowb0, ss0)
        scatter_wait(buf1, rowb1, ss1)
        scatter_wait(buf0, rowb0, ss0)

        plsc.subcore_barrier()

        @pl.when(sid < WB_TILES)
        def _():
            sl = pl.ds(sid * WB_ROWS, WB_ROWS)
            pltpu.sync_copy(acc.at[sl], out_h.at[cid, sl])

    return k(packed3, val3, embeds, zeros_blk)


def _tc_add(partials):
    def body(p_ref, o_ref):
        o_ref[...] = p_ref[0] + p_ref[1]

    return pl.pallas_call(
        body,
        out_shape=jax.ShapeDtypeStruct((N, D), jnp.float32),
        grid=(10,),
        in_specs=[pl.BlockSpec((NC, N // 10, D), lambda i: (0, i, 0))],
        out_specs=pl.BlockSpec((N // 10, D), lambda i: (i, 0)),
    )(partials)


def kernel(edge_index, edge_values, embeds):
    row = edge_index[0].astype(jnp.int32)
    col = edge_index[1].astype(jnp.int32)
    packed3 = ((row << 16) | col).reshape(NW, E_PER_W)
    val3 = edge_values.reshape(NW, E_PER_W)
    zeros_blk = jnp.zeros((WB_ROWS, D), jnp.float32)
    partials = _sc_spmm(packed3, val3, embeds, zeros_blk)
    return _tc_add(partials)
